# baseline (speedup 1.0000x reference)
import functools

import jax
import jax.numpy as jnp
from jax import lax
from jax.experimental import pallas as pl
from jax.experimental.pallas import tpu as pltpu
from jax.experimental.pallas import tpu_sc as plsc

VOCAB = 100000
EMBED = 64
BATCH = 1024
CTX = 20

NC = 2
NS = 16
NW = NC * NS
ROWS_PER_W = BATCH // NW
IDX_PER_W = ROWS_PER_W * CTX
GCHUNK = 128
NCHUNK = IDX_PER_W // GCHUNK


def _sc_gather_sum(idx_flat, emb_table):
    mesh = plsc.VectorSubcoreMesh(core_axis_name="c", subcore_axis_name="s")

    @functools.partial(
        pl.kernel,
        mesh=mesh,
        compiler_params=pltpu.CompilerParams(use_tc_tiling_on_sc=False),
        out_type=jax.ShapeDtypeStruct((BATCH, EMBED), jnp.float32),
        scratch_types=[
            pltpu.VMEM((IDX_PER_W,), jnp.int32),
            pltpu.VMEM((IDX_PER_W, EMBED), jnp.float32),
            pltpu.VMEM((ROWS_PER_W, EMBED), jnp.float32),
            pltpu.SemaphoreType.DMA,
        ],
    )
    def k(idx_hbm, table_hbm, out_hbm, idx_v, rows_v, out_v, sem):
        wid = lax.axis_index("s") * NC + lax.axis_index("c")
        base = wid * IDX_PER_W
        pltpu.sync_copy(idx_hbm.at[pl.ds(base, IDX_PER_W)], idx_v)
        copies = [
            pltpu.async_copy(
                table_hbm.at[idx_v.at[pl.ds(kk * GCHUNK, GCHUNK)]],
                rows_v.at[pl.ds(kk * GCHUNK, GCHUNK)],
                sem,
            )
            for kk in range(NCHUNK)
        ]
        for c in copies:
            c.wait()

        def body(bb, carry):
            for j in range(EMBED // 16):
                acc = rows_v[bb * CTX, pl.ds(j * 16, 16)]
                for cc in range(1, CTX):
                    acc = acc + rows_v[bb * CTX + cc, pl.ds(j * 16, 16)]
                out_v[bb, pl.ds(j * 16, 16)] = acc
            return carry

        lax.fori_loop(0, ROWS_PER_W, body, 0, unroll=False)
        pltpu.sync_copy(out_v, out_hbm.at[pl.ds(wid * ROWS_PER_W, ROWS_PER_W)])

    return k(idx_flat, emb_table)


VB = 1024
NV = (VOCAB + VB - 1) // VB


def _lse_kernel(s_ref, w_ref, b_ref, lse_ref, m_sc, s_sc):
    j = pl.program_id(0)
    x = lax.dot_general(
        s_ref[...].astype(jnp.bfloat16),
        w_ref[...].astype(jnp.bfloat16),
        (((1,), (1,)), ((), ())),
        preferred_element_type=jnp.float32,
    )
    x = x + b_ref[...]
    col = j * VB + lax.broadcasted_iota(jnp.int32, x.shape, 1)
    x = jnp.where(col < VOCAB, x, -1e30)
    bm = jnp.max(x, axis=1, keepdims=True)

    @pl.when(j == 0)
    def _():
        m_sc[...] = bm
        s_sc[...] = jnp.sum(jnp.exp(x - bm), axis=1, keepdims=True)

    @pl.when(j > 0)
    def _():
        m_prev = m_sc[...]
        m_new = jnp.maximum(m_prev, bm)
        s_sc[...] = s_sc[...] * jnp.exp(m_prev - m_new) + jnp.sum(
            jnp.exp(x - m_new), axis=1, keepdims=True
        )
        m_sc[...] = m_new

    @pl.when(j == NV - 1)
    def _():
        lse_ref[...] = m_sc[...] + jnp.log(s_sc[...])


def _out_kernel(s_ref, w_ref, b_ref, lse_ref, o_ref):
    x = lax.dot_general(
        s_ref[...].astype(jnp.bfloat16),
        w_ref[...].astype(jnp.bfloat16),
        (((1,), (1,)), ((), ())),
        preferred_element_type=jnp.float32,
    )
    o_ref[...] = x + b_ref[...] - lse_ref[...]


def _tc_log_softmax(summed, W, b2):
    lse = pl.pallas_call(
        _lse_kernel,
        grid=(NV,),
        in_specs=[
            pl.BlockSpec((BATCH, EMBED), lambda j: (0, 0)),
            pl.BlockSpec((VB, EMBED), lambda j: (j, 0)),
            pl.BlockSpec((1, VB), lambda j: (0, j)),
        ],
        out_specs=pl.BlockSpec((BATCH, 1), lambda j: (0, 0)),
        out_shape=jax.ShapeDtypeStruct((BATCH, 1), jnp.float32),
        scratch_shapes=[
            pltpu.VMEM((BATCH, 1), jnp.float32),
            pltpu.VMEM((BATCH, 1), jnp.float32),
        ],
    )(summed, W, b2)

    out = pl.pallas_call(
        _out_kernel,
        grid=(NV,),
        in_specs=[
            pl.BlockSpec((BATCH, EMBED), lambda j: (0, 0)),
            pl.BlockSpec((VB, EMBED), lambda j: (j, 0)),
            pl.BlockSpec((1, VB), lambda j: (0, j)),
            pl.BlockSpec((BATCH, 1), lambda j: (0, 0)),
        ],
        out_specs=pl.BlockSpec((BATCH, VB), lambda j: (0, j)),
        out_shape=jax.ShapeDtypeStruct((BATCH, VOCAB), jnp.float32),
    )(summed, W, b2, lse)
    return out


def kernel(inputs, emb_table, W, b):
    idx_flat = inputs.reshape(-1)
    summed = _sc_gather_sum(idx_flat, emb_table)
    b2 = b.reshape(1, VOCAB)
    return _tc_log_softmax(summed, W, b2)


# speedup vs baseline: 1.0561x; 1.0561x over previous
import functools

import jax
import jax.numpy as jnp
from jax import lax
from jax.experimental import pallas as pl
from jax.experimental.pallas import tpu as pltpu
from jax.experimental.pallas import tpu_sc as plsc

VOCAB = 100000
EMBED = 64
BATCH = 1024
CTX = 20

NC = 2
NS = 16
NW = NC * NS
ROWS_PER_W = BATCH // NW
IDX_PER_W = ROWS_PER_W * CTX
GCHUNK = 128
NCHUNK = IDX_PER_W // GCHUNK


def _sc_gather_sum(idx_flat, emb_table):
    mesh = plsc.VectorSubcoreMesh(core_axis_name="c", subcore_axis_name="s")

    @functools.partial(
        pl.kernel,
        mesh=mesh,
        compiler_params=pltpu.CompilerParams(use_tc_tiling_on_sc=False),
        out_type=jax.ShapeDtypeStruct((BATCH, EMBED), jnp.float32),
        scratch_types=[
            pltpu.VMEM((IDX_PER_W,), jnp.int32),
            pltpu.VMEM((IDX_PER_W, EMBED), jnp.float32),
            pltpu.VMEM((ROWS_PER_W, EMBED), jnp.float32),
            pltpu.SemaphoreType.DMA,
        ],
    )
    def k(idx_hbm, table_hbm, out_hbm, idx_v, rows_v, out_v, sem):
        wid = lax.axis_index("s") * NC + lax.axis_index("c")
        base = wid * IDX_PER_W
        pltpu.sync_copy(idx_hbm.at[pl.ds(base, IDX_PER_W)], idx_v)
        copies = [
            pltpu.async_copy(
                table_hbm.at[idx_v.at[pl.ds(kk * GCHUNK, GCHUNK)]],
                rows_v.at[pl.ds(kk * GCHUNK, GCHUNK)],
                sem,
            )
            for kk in range(NCHUNK)
        ]
        for c in copies:
            c.wait()

        def body(bb, carry):
            for j in range(EMBED // 16):
                acc = rows_v[bb * CTX, pl.ds(j * 16, 16)]
                for cc in range(1, CTX):
                    acc = acc + rows_v[bb * CTX + cc, pl.ds(j * 16, 16)]
                out_v[bb, pl.ds(j * 16, 16)] = acc
            return carry

        lax.fori_loop(0, ROWS_PER_W, body, 0, unroll=False)
        pltpu.sync_copy(out_v, out_hbm.at[pl.ds(wid * ROWS_PER_W, ROWS_PER_W)])

    return k(idx_flat, emb_table)


VB = 1024
NV = (VOCAB + VB - 1) // VB


def _lse_kernel(s_ref, w_ref, b_ref, lse_ref, m_sc, s_sc):
    j = pl.program_id(0)
    x = lax.dot_general(
        s_ref[...],
        w_ref[...],
        (((1,), (0,)), ((), ())),
        preferred_element_type=jnp.float32,
    )
    x = x + b_ref[...]
    col = j * VB + lax.broadcasted_iota(jnp.int32, x.shape, 1)
    x = jnp.where(col < VOCAB, x, -1e30)
    bm = jnp.max(x, axis=1, keepdims=True)

    @pl.when(j == 0)
    def _():
        m_sc[...] = bm
        s_sc[...] = jnp.sum(jnp.exp(x - bm), axis=1, keepdims=True)

    @pl.when(j > 0)
    def _():
        m_prev = m_sc[...]
        m_new = jnp.maximum(m_prev, bm)
        s_sc[...] = s_sc[...] * jnp.exp(m_prev - m_new) + jnp.sum(
            jnp.exp(x - m_new), axis=1, keepdims=True
        )
        m_sc[...] = m_new

    @pl.when(j == NV - 1)
    def _():
        lse_ref[...] = m_sc[...] + jnp.log(s_sc[...])


def _out_kernel(s_ref, w_ref, b_ref, lse_ref, o_ref):
    x = lax.dot_general(
        s_ref[...],
        w_ref[...],
        (((1,), (0,)), ((), ())),
        preferred_element_type=jnp.float32,
    )
    o_ref[...] = x + b_ref[...] - lse_ref[...]


def _tc_log_softmax(s16, wt16, b2):
    lse = pl.pallas_call(
        _lse_kernel,
        grid=(NV,),
        in_specs=[
            pl.BlockSpec((BATCH, EMBED), lambda j: (0, 0)),
            pl.BlockSpec((EMBED, VB), lambda j: (0, j)),
            pl.BlockSpec((1, VB), lambda j: (0, j)),
        ],
        out_specs=pl.BlockSpec((BATCH, 1), lambda j: (0, 0)),
        out_shape=jax.ShapeDtypeStruct((BATCH, 1), jnp.float32),
        scratch_shapes=[
            pltpu.VMEM((BATCH, 1), jnp.float32),
            pltpu.VMEM((BATCH, 1), jnp.float32),
        ],
    )(s16, wt16, b2)

    out = pl.pallas_call(
        _out_kernel,
        grid=(NV,),
        in_specs=[
            pl.BlockSpec((BATCH, EMBED), lambda j: (0, 0)),
            pl.BlockSpec((EMBED, VB), lambda j: (0, j)),
            pl.BlockSpec((1, VB), lambda j: (0, j)),
            pl.BlockSpec((BATCH, 1), lambda j: (0, 0)),
        ],
        out_specs=pl.BlockSpec((BATCH, VB), lambda j: (0, j)),
        out_shape=jax.ShapeDtypeStruct((BATCH, VOCAB), jnp.float32),
    )(s16, wt16, b2, lse)
    return out


def kernel(inputs, emb_table, W, b):
    idx_flat = inputs.reshape(-1)
    summed = _sc_gather_sum(idx_flat, emb_table)
    s16 = summed.astype(jnp.bfloat16)
    wt16 = W.T.astype(jnp.bfloat16)
    b2 = b.reshape(1, VOCAB)
    return _tc_log_softmax(s16, wt16, b2)


# speedup vs baseline: 1.1174x; 1.0581x over previous
import functools

import jax
import jax.numpy as jnp
from jax import lax
from jax.experimental import pallas as pl
from jax.experimental.pallas import tpu as pltpu
from jax.experimental.pallas import tpu_sc as plsc

VOCAB = 100000
EMBED = 64
BATCH = 1024
CTX = 20

NC = 2
NS = 16
NW = NC * NS
ROWS_PER_W = BATCH // NW
IDX_PER_W = ROWS_PER_W * CTX
GCHUNK = 128
NCHUNK = IDX_PER_W // GCHUNK


def _sc_gather_sum(idx_flat, emb_table):
    mesh = plsc.VectorSubcoreMesh(core_axis_name="c", subcore_axis_name="s")

    @functools.partial(
        pl.kernel,
        mesh=mesh,
        compiler_params=pltpu.CompilerParams(use_tc_tiling_on_sc=False),
        out_type=jax.ShapeDtypeStruct((BATCH, EMBED), jnp.float32),
        scratch_types=[
            pltpu.VMEM((IDX_PER_W,), jnp.int32),
            pltpu.VMEM((IDX_PER_W, EMBED), jnp.float32),
            pltpu.VMEM((ROWS_PER_W, EMBED), jnp.float32),
            pltpu.SemaphoreType.DMA,
        ],
    )
    def k(idx_hbm, table_hbm, out_hbm, idx_v, rows_v, out_v, sem):
        wid = lax.axis_index("s") * NC + lax.axis_index("c")
        base = wid * IDX_PER_W
        pltpu.sync_copy(idx_hbm.at[pl.ds(base, IDX_PER_W)], idx_v)
        copies = [
            pltpu.async_copy(
                table_hbm.at[idx_v.at[pl.ds(kk * GCHUNK, GCHUNK)]],
                rows_v.at[pl.ds(kk * GCHUNK, GCHUNK)],
                sem,
            )
            for kk in range(NCHUNK)
        ]
        for c in copies:
            c.wait()

        def body(bb, carry):
            for j in range(EMBED // 16):
                acc = rows_v[bb * CTX, pl.ds(j * 16, 16)]
                for cc in range(1, CTX):
                    acc = acc + rows_v[bb * CTX + cc, pl.ds(j * 16, 16)]
                out_v[bb, pl.ds(j * 16, 16)] = acc
            return carry

        lax.fori_loop(0, ROWS_PER_W, body, 0, unroll=False)
        pltpu.sync_copy(out_v, out_hbm.at[pl.ds(wid * ROWS_PER_W, ROWS_PER_W)])

    return k(idx_flat, emb_table)


VB = 2048
NV = (VOCAB + VB - 1) // VB
VP = NV * VB


def _lse_kernel(s_ref, w_ref, b_ref, lse_ref, m_sc, s_sc):
    j = pl.program_id(0)
    x = lax.dot_general(
        s_ref[...],
        w_ref[...],
        (((1,), (0,)), ((), ())),
        preferred_element_type=jnp.float32,
    )
    x = x + b_ref[...]
    bm = jnp.max(x, axis=1, keepdims=True)

    @pl.when(j == 0)
    def _():
        m_sc[...] = bm
        s_sc[...] = jnp.sum(jnp.exp(x - bm), axis=1, keepdims=True)

    @pl.when(j > 0)
    def _():
        m_prev = m_sc[...]
        m_new = jnp.maximum(m_prev, bm)
        s_sc[...] = s_sc[...] * jnp.exp(m_prev - m_new) + jnp.sum(
            jnp.exp(x - m_new), axis=1, keepdims=True
        )
        m_sc[...] = m_new

    @pl.when(j == NV - 1)
    def _():
        lse_ref[...] = m_sc[...] + jnp.log(s_sc[...])


def _out_kernel(s_ref, w_ref, b_ref, lse_ref, o_ref):
    x = lax.dot_general(
        s_ref[...],
        w_ref[...],
        (((1,), (0,)), ((), ())),
        preferred_element_type=jnp.float32,
    )
    o_ref[...] = x + b_ref[...] - lse_ref[...]


def _tc_log_softmax(s16, wt16, b2):
    lse = pl.pallas_call(
        _lse_kernel,
        grid=(NV,),
        in_specs=[
            pl.BlockSpec((BATCH, EMBED), lambda j: (0, 0)),
            pl.BlockSpec((EMBED, VB), lambda j: (0, j)),
            pl.BlockSpec((1, VB), lambda j: (0, j)),
        ],
        out_specs=pl.BlockSpec((BATCH, 1), lambda j: (0, 0)),
        out_shape=jax.ShapeDtypeStruct((BATCH, 1), jnp.float32),
        scratch_shapes=[
            pltpu.VMEM((BATCH, 1), jnp.float32),
            pltpu.VMEM((BATCH, 1), jnp.float32),
        ],
    )(s16, wt16, b2)

    out = pl.pallas_call(
        _out_kernel,
        grid=(NV,),
        in_specs=[
            pl.BlockSpec((BATCH, EMBED), lambda j: (0, 0)),
            pl.BlockSpec((EMBED, VB), lambda j: (0, j)),
            pl.BlockSpec((1, VB), lambda j: (0, j)),
            pl.BlockSpec((BATCH, 1), lambda j: (0, 0)),
        ],
        out_specs=pl.BlockSpec((BATCH, VB), lambda j: (0, j)),
        out_shape=jax.ShapeDtypeStruct((BATCH, VOCAB), jnp.float32),
    )(s16, wt16, b2, lse)
    return out


def kernel(inputs, emb_table, W, b):
    idx_flat = inputs.reshape(-1)
    summed = _sc_gather_sum(idx_flat, emb_table)
    s16 = summed.astype(jnp.bfloat16)
    wt16 = jnp.pad(W.T.astype(jnp.bfloat16), ((0, 0), (0, VP - VOCAB)))
    b2 = jnp.pad(b, (0, VP - VOCAB), constant_values=-1e30).reshape(1, VP)
    return _tc_log_softmax(s16, wt16, b2)
